# Initial kernel scaffold; baseline (speedup 1.0000x reference)
import jax
import jax.numpy as jnp
from jax.experimental import pallas as pl


def kernel(x, edge_index, W1, b1, W2, b2, W3, b3, W4, b4, W5, b5):
    raise NotImplementedError("write your pallas kernel here")


# speedup vs baseline: 18.1373x; 18.1373x over previous
import functools

import jax
import jax.numpy as jnp
from jax import lax
from jax.experimental import pallas as pl
from jax.experimental.pallas import tpu as pltpu
from jax.experimental.pallas import tpu_sc as plsc

N = 10000
E = 160000
NC = 2
NS = 16
NW = NC * NS
PW = E // NW
CB = 125
CH = PW // CB
STRIPE = N // NS

f32 = jnp.float32


def _make_agg(fo, gather):
    mesh = plsc.VectorSubcoreMesh(core_axis_name="c", subcore_axis_name="s")

    @functools.partial(
        pl.kernel,
        out_type=jax.ShapeDtypeStruct((NC, N, fo), f32),
        mesh=mesh,
        compiler_params=pltpu.CompilerParams(use_tc_tiling_on_sc=False),
        scratch_types=[
            pltpu.VMEM((CH, CB), jnp.int32),
            pltpu.VMEM((CH, CB), jnp.int32),
            pltpu.VMEM((CB, fo), f32),
            pltpu.VMEM((STRIPE, fo), f32),
            pltpu.VMEM_SHARED((N, fo), f32),
            pltpu.SemaphoreType.DMA,
        ],
    )
    def agg(hs_hbm, src_hbm, dst_hbm, out_hbm, src_v, dst_v, rows_v, tmp_v,
            acc, sem):
        c = lax.axis_index("c")
        s = lax.axis_index("s")
        wid = s * NC + c

        zero16 = jnp.zeros((16,), f32)

        def zrow(r, carry):
            for cc in range(fo // 16):
                tmp_v[r, pl.ds(cc * 16, 16)] = zero16
            return carry

        lax.fori_loop(0, STRIPE, zrow, 0)
        pltpu.sync_copy(tmp_v, acc.at[pl.ds(s * STRIPE, STRIPE)])

        pltpu.sync_copy(src_hbm.at[wid], src_v)
        pltpu.sync_copy(dst_hbm.at[wid], dst_v)
        if not gather:
            one16 = jnp.ones((16,), f32)

            def orow(r, carry):
                for cc in range(fo // 16):
                    rows_v[r, pl.ds(cc * 16, 16)] = one16
                return carry

            lax.fori_loop(0, CB, orow, 0)
        plsc.subcore_barrier()

        def chunk(j, carry):
            if gather:
                pltpu.async_copy(hs_hbm.at[src_v.at[j]], rows_v, sem).wait()
            pltpu.sync_copy(rows_v, acc.at[dst_v.at[j]], add=True)
            return carry

        lax.fori_loop(0, CH, chunk, 0)
        plsc.subcore_barrier()

        pltpu.sync_copy(acc.at[pl.ds(s * STRIPE, STRIPE)], tmp_v)
        pltpu.sync_copy(tmp_v, out_hbm.at[c, pl.ds(s * STRIPE, STRIPE)])

    return agg


BN = 2000


def _tc_first(x, W1, degp):
    def body(x_ref, w_ref, degp_ref, dinv_ref, hs_ref):
        deg = degp_ref[0][:, 0:1] + degp_ref[1][:, 0:1] + 1.0
        dinv = lax.rsqrt(deg)
        h = jnp.dot(x_ref[...], w_ref[...], preferred_element_type=f32)
        dinv_ref[...] = dinv
        hs_ref[...] = h * dinv

    return pl.pallas_call(
        body,
        grid=(N // BN,),
        in_specs=[
            pl.BlockSpec((BN, 500), lambda i: (i, 0)),
            pl.BlockSpec((500, 64), lambda i: (0, 0)),
            pl.BlockSpec((NC, BN, 16), lambda i: (0, i, 0)),
        ],
        out_specs=[
            pl.BlockSpec((BN, 1), lambda i: (i, 0)),
            pl.BlockSpec((BN, 64), lambda i: (i, 0)),
        ],
        out_shape=[
            jax.ShapeDtypeStruct((N, 1), f32),
            jax.ShapeDtypeStruct((N, 64), f32),
        ],
    )(x, W1, degp)


def _tc_mid(p, hs, dinv, b, Wn, relu, leaf_cols, leaf_from_act=False):
    fo = hs.shape[1]
    fn = Wn.shape[1]

    def body(p_ref, hs_ref, dinv_ref, b_ref, w_ref, *out_refs):
        dinv = dinv_ref[...]
        out = dinv * (p_ref[0] + p_ref[1] + hs_ref[...]) + b_ref[...]
        act = jnp.maximum(out, 0.0) if relu else out
        if leaf_cols:
            leaf = act if leaf_from_act else out
            out_refs[0][...] = leaf[:, :leaf_cols]
        h = jnp.dot(act, w_ref[...], preferred_element_type=f32)
        out_refs[-1][...] = h * dinv

    out_specs = [pl.BlockSpec((BN, fn), lambda i: (i, 0))]
    out_shape = [jax.ShapeDtypeStruct((N, fn), f32)]
    if leaf_cols:
        out_specs.insert(0, pl.BlockSpec((BN, leaf_cols), lambda i: (i, 0)))
        out_shape.insert(0, jax.ShapeDtypeStruct((N, leaf_cols), f32))

    res = pl.pallas_call(
        body,
        grid=(N // BN,),
        in_specs=[
            pl.BlockSpec((NC, BN, fo), lambda i: (0, i, 0)),
            pl.BlockSpec((BN, fo), lambda i: (i, 0)),
            pl.BlockSpec((BN, 1), lambda i: (i, 0)),
            pl.BlockSpec((1, fo), lambda i: (0, 0)),
            pl.BlockSpec(Wn.shape, lambda i: (0, 0)),
        ],
        out_specs=out_specs,
        out_shape=out_shape,
    )(p, hs, dinv, b, Wn)
    return res if leaf_cols else (None, res[0])


def _tc_last(p, hs, dinv, b):
    fo = hs.shape[1]

    def body(p_ref, hs_ref, dinv_ref, b_ref, lp_ref):
        out = dinv_ref[...] * (p_ref[0] + p_ref[1] + hs_ref[...]) + b_ref[...]
        x9 = out[:, :3]
        m = jnp.max(x9, axis=1, keepdims=True)
        e = jnp.exp(x9 - m)
        lp_ref[...] = x9 - m - jnp.log(jnp.sum(e, axis=1, keepdims=True))

    return pl.pallas_call(
        body,
        grid=(N // BN,),
        in_specs=[
            pl.BlockSpec((NC, BN, fo), lambda i: (0, i, 0)),
            pl.BlockSpec((BN, fo), lambda i: (i, 0)),
            pl.BlockSpec((BN, 1), lambda i: (i, 0)),
            pl.BlockSpec((1, fo), lambda i: (0, 0)),
        ],
        out_specs=pl.BlockSpec((BN, 3), lambda i: (i, 0)),
        out_shape=jax.ShapeDtypeStruct((N, 3), f32),
    )(p, hs, dinv, b)


def kernel(x, edge_index, W1, b1, W2, b2, W3, b3, W4, b4, W5, b5):
    src3 = edge_index[0].reshape(NW, CH, CB)
    dst3 = edge_index[1].reshape(NW, CH, CB)

    W3p = jnp.pad(W3, ((0, 0), (0, 8)))
    b3p = jnp.pad(b3, (0, 8))
    W4p = jnp.pad(W4, ((0, 8), (0, 0)))
    W5p = jnp.pad(W5, ((0, 0), (0, 13)))
    b5p = jnp.pad(b5, (0, 13))
    dummy = jnp.zeros((N, 16), f32)

    degp = _make_agg(16, gather=False)(dummy, src3, dst3)
    dinv, hs1 = _tc_first(x, W1, degp)

    p1 = _make_agg(64, gather=True)(hs1, src3, dst3)
    x2, hs2 = _tc_mid(p1, hs1, dinv, b1.reshape(1, -1), W2,
                      relu=True, leaf_cols=64, leaf_from_act=True)

    p2 = _make_agg(32, gather=True)(hs2, src3, dst3)
    _, hs3 = _tc_mid(p2, hs2, dinv, b2.reshape(1, -1), W3p,
                     relu=True, leaf_cols=0)

    p3 = _make_agg(32, gather=True)(hs3, src3, dst3)
    x5, hs4 = _tc_mid(p3, hs3, dinv, b3p.reshape(1, -1), W4p,
                      relu=True, leaf_cols=24)

    p4 = _make_agg(16, gather=True)(hs4, src3, dst3)
    x8, hs5 = _tc_mid(p4, hs4, dinv, b4.reshape(1, -1), W5p,
                      relu=False, leaf_cols=16)

    p5 = _make_agg(16, gather=True)(hs5, src3, dst3)
    logp = _tc_last(p5, hs5, dinv, b5p.reshape(1, -1))

    return (logp, x2, x5, x8)


# speedup vs baseline: 24.4092x; 1.3458x over previous
import functools

import jax
import jax.numpy as jnp
from jax import lax
from jax.experimental import pallas as pl
from jax.experimental.pallas import tpu as pltpu
from jax.experimental.pallas import tpu_sc as plsc

N = 10000
E = 160000
NC = 2
NS = 16
NW = NC * NS
PW = E // NW
CB = 125
CH = PW // CB
STRIPE = N // NS

f32 = jnp.float32


NB = 4
NIT = CH // NB


def _make_agg(fo, gather):
    mesh = plsc.VectorSubcoreMesh(core_axis_name="c", subcore_axis_name="s")

    @functools.partial(
        pl.kernel,
        out_type=jax.ShapeDtypeStruct((NC, N, fo), f32),
        mesh=mesh,
        compiler_params=pltpu.CompilerParams(use_tc_tiling_on_sc=False),
        scratch_types=[
            pltpu.VMEM((CH, CB), jnp.int32),
            pltpu.VMEM((CH, CB), jnp.int32),
            pltpu.VMEM((NB, CB, fo), f32),
            pltpu.VMEM((STRIPE, fo), f32),
            pltpu.VMEM_SHARED((N, fo), f32),
            [pltpu.SemaphoreType.DMA] * NB,
            [pltpu.SemaphoreType.DMA] * NB,
        ],
    )
    def agg(hs_hbm, src_hbm, dst_hbm, out_hbm, src_v, dst_v, rows_v, tmp_v,
            acc, sem_g, sem_s):
        c = lax.axis_index("c")
        s = lax.axis_index("s")
        wid = s * NC + c

        zero16 = jnp.zeros((16,), f32)

        def zrow(r, carry):
            for cc in range(fo // 16):
                tmp_v[r, pl.ds(cc * 16, 16)] = zero16
            return carry

        lax.fori_loop(0, STRIPE, zrow, 0)
        pltpu.sync_copy(tmp_v, acc.at[pl.ds(s * STRIPE, STRIPE)])

        pltpu.sync_copy(src_hbm.at[wid], src_v)
        pltpu.sync_copy(dst_hbm.at[wid], dst_v)
        plsc.subcore_barrier()

        if gather:
            for b in range(NB):
                pltpu.async_copy(hs_hbm.at[src_v.at[b]], rows_v.at[b],
                                 sem_g[b])

            def ring(it, carry):
                for b in range(NB):
                    j = it * NB + b
                    pltpu.make_async_copy(hs_hbm.at[src_v.at[j]],
                                          rows_v.at[b], sem_g[b]).wait()
                    pltpu.async_copy(rows_v.at[b], acc.at[dst_v.at[j]],
                                     sem_s[b], add=True)
                for b in range(NB):
                    j = it * NB + b
                    pltpu.make_async_copy(rows_v.at[b], acc.at[dst_v.at[j]],
                                          sem_s[b]).wait()

                    @pl.when(it < NIT - 1)
                    def _():
                        pltpu.async_copy(hs_hbm.at[src_v.at[j + NB]],
                                         rows_v.at[b], sem_g[b])
                return carry

            lax.fori_loop(0, NIT, ring, 0)
        else:
            one16 = jnp.ones((16,), f32)

            def orow(r, carry):
                for cc in range(fo // 16):
                    rows_v[0, r, pl.ds(cc * 16, 16)] = one16
                return carry

            lax.fori_loop(0, CB, orow, 0)

            for b in range(NB):
                pltpu.async_copy(rows_v.at[0], acc.at[dst_v.at[b]],
                                 sem_s[b], add=True)

            def fire(it, carry):
                for b in range(NB):
                    j = it * NB + b
                    pltpu.make_async_copy(rows_v.at[0], acc.at[dst_v.at[j]],
                                          sem_s[b]).wait()

                    @pl.when(it < NIT - 1)
                    def _():
                        pltpu.async_copy(rows_v.at[0],
                                         acc.at[dst_v.at[j + NB]],
                                         sem_s[b], add=True)
                return carry

            lax.fori_loop(0, NIT, fire, 0)

        plsc.subcore_barrier()
        pltpu.sync_copy(acc.at[pl.ds(s * STRIPE, STRIPE)], tmp_v)
        pltpu.sync_copy(tmp_v, out_hbm.at[c, pl.ds(s * STRIPE, STRIPE)])

    return agg


BN = 2000


def _tc_matmul1(x, W1):
    def body(x_ref, w_ref, h_ref):
        h_ref[...] = jnp.dot(x_ref[...], w_ref[...],
                             preferred_element_type=f32)

    return pl.pallas_call(
        body,
        grid=(N // BN,),
        in_specs=[
            pl.BlockSpec((BN, 500), lambda i: (i, 0)),
            pl.BlockSpec((500, 64), lambda i: (0, 0)),
        ],
        out_specs=pl.BlockSpec((BN, 64), lambda i: (i, 0)),
        out_shape=jax.ShapeDtypeStruct((N, 64), f32),
    )(x, W1)


def _tc_first(h1, degp):
    def body(h_ref, degp_ref, dinv_ref, hs_ref):
        deg = degp_ref[0][:, 0:1] + degp_ref[1][:, 0:1] + 1.0
        dinv = lax.rsqrt(deg)
        dinv_ref[...] = dinv
        hs_ref[...] = h_ref[...] * dinv

    return pl.pallas_call(
        body,
        grid=(N // BN,),
        in_specs=[
            pl.BlockSpec((BN, 64), lambda i: (i, 0)),
            pl.BlockSpec((NC, BN, 16), lambda i: (0, i, 0)),
        ],
        out_specs=[
            pl.BlockSpec((BN, 1), lambda i: (i, 0)),
            pl.BlockSpec((BN, 64), lambda i: (i, 0)),
        ],
        out_shape=[
            jax.ShapeDtypeStruct((N, 1), f32),
            jax.ShapeDtypeStruct((N, 64), f32),
        ],
    )(h1, degp)


def _tc_mid(p, hs, dinv, b, Wn, relu, leaf_cols, leaf_from_act=False):
    fo = hs.shape[1]
    fn = Wn.shape[1]

    def body(p_ref, hs_ref, dinv_ref, b_ref, w_ref, *out_refs):
        dinv = dinv_ref[...]
        out = dinv * (p_ref[0] + p_ref[1] + hs_ref[...]) + b_ref[...]
        act = jnp.maximum(out, 0.0) if relu else out
        if leaf_cols:
            leaf = act if leaf_from_act else out
            out_refs[0][...] = leaf[:, :leaf_cols]
        h = jnp.dot(act, w_ref[...], preferred_element_type=f32)
        out_refs[-1][...] = h * dinv

    out_specs = [pl.BlockSpec((BN, fn), lambda i: (i, 0))]
    out_shape = [jax.ShapeDtypeStruct((N, fn), f32)]
    if leaf_cols:
        out_specs.insert(0, pl.BlockSpec((BN, leaf_cols), lambda i: (i, 0)))
        out_shape.insert(0, jax.ShapeDtypeStruct((N, leaf_cols), f32))

    res = pl.pallas_call(
        body,
        grid=(N // BN,),
        in_specs=[
            pl.BlockSpec((NC, BN, fo), lambda i: (0, i, 0)),
            pl.BlockSpec((BN, fo), lambda i: (i, 0)),
            pl.BlockSpec((BN, 1), lambda i: (i, 0)),
            pl.BlockSpec((1, fo), lambda i: (0, 0)),
            pl.BlockSpec(Wn.shape, lambda i: (0, 0)),
        ],
        out_specs=out_specs,
        out_shape=out_shape,
    )(p, hs, dinv, b, Wn)
    return res if leaf_cols else (None, res[0])


def _tc_last(p, hs, dinv, b):
    fo = hs.shape[1]

    def body(p_ref, hs_ref, dinv_ref, b_ref, lp_ref):
        out = dinv_ref[...] * (p_ref[0] + p_ref[1] + hs_ref[...]) + b_ref[...]
        x9 = out[:, :3]
        m = jnp.max(x9, axis=1, keepdims=True)
        e = jnp.exp(x9 - m)
        lp_ref[...] = x9 - m - jnp.log(jnp.sum(e, axis=1, keepdims=True))

    return pl.pallas_call(
        body,
        grid=(N // BN,),
        in_specs=[
            pl.BlockSpec((NC, BN, fo), lambda i: (0, i, 0)),
            pl.BlockSpec((BN, fo), lambda i: (i, 0)),
            pl.BlockSpec((BN, 1), lambda i: (i, 0)),
            pl.BlockSpec((1, fo), lambda i: (0, 0)),
        ],
        out_specs=pl.BlockSpec((BN, 3), lambda i: (i, 0)),
        out_shape=jax.ShapeDtypeStruct((N, 3), f32),
    )(p, hs, dinv, b)


def kernel(x, edge_index, W1, b1, W2, b2, W3, b3, W4, b4, W5, b5):
    src3 = edge_index[0].reshape(NW, CH, CB)
    dst3 = edge_index[1].reshape(NW, CH, CB)

    W3p = jnp.pad(W3, ((0, 0), (0, 8)))
    b3p = jnp.pad(b3, (0, 8))
    W4p = jnp.pad(W4, ((0, 8), (0, 0)))
    W5p = jnp.pad(W5, ((0, 0), (0, 13)))
    b5p = jnp.pad(b5, (0, 13))
    dummy = jnp.zeros((N, 16), f32)

    h1 = _tc_matmul1(x, W1)
    degp = _make_agg(16, gather=False)(dummy, src3, dst3)
    dinv, hs1 = _tc_first(h1, degp)

    p1 = _make_agg(64, gather=True)(hs1, src3, dst3)
    x2, hs2 = _tc_mid(p1, hs1, dinv, b1.reshape(1, -1), W2,
                      relu=True, leaf_cols=64, leaf_from_act=True)

    p2 = _make_agg(32, gather=True)(hs2, src3, dst3)
    _, hs3 = _tc_mid(p2, hs2, dinv, b2.reshape(1, -1), W3p,
                     relu=True, leaf_cols=0)

    p3 = _make_agg(32, gather=True)(hs3, src3, dst3)
    x5, hs4 = _tc_mid(p3, hs3, dinv, b3p.reshape(1, -1), W4p,
                      relu=True, leaf_cols=24)

    p4 = _make_agg(16, gather=True)(hs4, src3, dst3)
    x8, hs5 = _tc_mid(p4, hs4, dinv, b4.reshape(1, -1), W5p,
                      relu=False, leaf_cols=16)

    p5 = _make_agg(16, gather=True)(hs5, src3, dst3)
    logp = _tc_last(p5, hs5, dinv, b5p.reshape(1, -1))

    return (logp, x2, x5, x8)


# speedup vs baseline: 25.0258x; 1.0253x over previous
import functools

import jax
import jax.numpy as jnp
from jax import lax
from jax.experimental import pallas as pl
from jax.experimental.pallas import tpu as pltpu
from jax.experimental.pallas import tpu_sc as plsc

N = 10000
E = 160000
NC = 2
NS = 16
NW = NC * NS
PW = E // NW
CB = 125
CH = PW // CB
SA = 640
SB = N - (NS - 1) * SA

f32 = jnp.float32


NB = 4
NIT = CH // NB


def _make_agg(fo, gather):
    mesh = plsc.VectorSubcoreMesh(core_axis_name="c", subcore_axis_name="s")

    @functools.partial(
        pl.kernel,
        out_type=jax.ShapeDtypeStruct((NC, N, fo), f32),
        mesh=mesh,
        compiler_params=pltpu.CompilerParams(use_tc_tiling_on_sc=False),
        scratch_types=[
            pltpu.VMEM((CH, CB), jnp.int32),
            pltpu.VMEM((CH, CB), jnp.int32),
            pltpu.VMEM((NB, CB, fo), f32),
            pltpu.VMEM_SHARED((N, fo), f32),
            [pltpu.SemaphoreType.DMA] * NB,
            [pltpu.SemaphoreType.DMA] * NB,
        ],
    )
    def agg(init_hbm, zeros_hbm, src_hbm, dst_hbm, out_hbm, src_v, dst_v,
            rows_v, acc, sem_g, sem_s):
        c = lax.axis_index("c")
        s = lax.axis_index("s")
        wid = s * NC + c
        off = pl.multiple_of(s * SA, 8)

        for cc, src_init in ((0, init_hbm), (1, zeros_hbm)):
            @pl.when(c == cc)
            def _():
                @pl.when(s < NS - 1)
                def _():
                    pltpu.sync_copy(src_init.at[pl.ds(off, SA)],
                                    acc.at[pl.ds(off, SA)])

                @pl.when(s == NS - 1)
                def _():
                    pltpu.sync_copy(src_init.at[pl.ds(off, SB)],
                                    acc.at[pl.ds(off, SB)])

        pltpu.sync_copy(src_hbm.at[wid], src_v)
        pltpu.sync_copy(dst_hbm.at[wid], dst_v)
        plsc.subcore_barrier()

        if gather:
            for b in range(NB):
                pltpu.async_copy(init_hbm.at[src_v.at[b]], rows_v.at[b],
                                 sem_g[b])

            def ring(it, carry):
                for b in range(NB):
                    j = it * NB + b
                    pltpu.make_async_copy(init_hbm.at[src_v.at[j]],
                                          rows_v.at[b], sem_g[b]).wait()
                    pltpu.async_copy(rows_v.at[b], acc.at[dst_v.at[j]],
                                     sem_s[b], add=True)
                for b in range(NB):
                    j = it * NB + b
                    pltpu.make_async_copy(rows_v.at[b], acc.at[dst_v.at[j]],
                                          sem_s[b]).wait()

                    @pl.when(it < NIT - 1)
                    def _():
                        pltpu.async_copy(init_hbm.at[src_v.at[j + NB]],
                                         rows_v.at[b], sem_g[b])
                return carry

            lax.fori_loop(0, NIT, ring, 0)
        else:
            one16 = jnp.ones((16,), f32)

            def orow(r, carry):
                for cc in range(fo // 16):
                    rows_v[0, r, pl.ds(cc * 16, 16)] = one16
                return carry

            lax.fori_loop(0, CB, orow, 0)

            for b in range(NB):
                pltpu.async_copy(rows_v.at[0], acc.at[dst_v.at[b]],
                                 sem_s[b], add=True)

            def fire(it, carry):
                for b in range(NB):
                    j = it * NB + b
                    pltpu.make_async_copy(rows_v.at[0], acc.at[dst_v.at[j]],
                                          sem_s[b]).wait()

                    @pl.when(it < NIT - 1)
                    def _():
                        pltpu.async_copy(rows_v.at[0],
                                         acc.at[dst_v.at[j + NB]],
                                         sem_s[b], add=True)
                return carry

            lax.fori_loop(0, NIT, fire, 0)

        plsc.subcore_barrier()

        @pl.when(s < NS - 1)
        def _():
            pltpu.sync_copy(acc.at[pl.ds(off, SA)],
                            out_hbm.at[c, pl.ds(off, SA)])

        @pl.when(s == NS - 1)
        def _():
            pltpu.sync_copy(acc.at[pl.ds(off, SB)],
                            out_hbm.at[c, pl.ds(off, SB)])

    return agg


BN = 2000


def _tc_matmul1(x, W1):
    def body(x_ref, w_ref, h_ref):
        h_ref[...] = jnp.dot(x_ref[...], w_ref[...],
                             preferred_element_type=f32)

    return pl.pallas_call(
        body,
        grid=(N // BN,),
        in_specs=[
            pl.BlockSpec((BN, 500), lambda i: (i, 0)),
            pl.BlockSpec((500, 64), lambda i: (0, 0)),
        ],
        out_specs=pl.BlockSpec((BN, 64), lambda i: (i, 0)),
        out_shape=jax.ShapeDtypeStruct((N, 64), f32),
    )(x, W1)


def _tc_first(h1, degp):
    def body(h_ref, degp_ref, dinv_ref, hs_ref):
        deg = degp_ref[0][:, 0:1] + degp_ref[1][:, 0:1]
        dinv = lax.rsqrt(deg)
        dinv_ref[...] = dinv
        hs_ref[...] = h_ref[...] * dinv

    return pl.pallas_call(
        body,
        grid=(N // BN,),
        in_specs=[
            pl.BlockSpec((BN, 64), lambda i: (i, 0)),
            pl.BlockSpec((NC, BN, 16), lambda i: (0, i, 0)),
        ],
        out_specs=[
            pl.BlockSpec((BN, 1), lambda i: (i, 0)),
            pl.BlockSpec((BN, 64), lambda i: (i, 0)),
        ],
        out_shape=[
            jax.ShapeDtypeStruct((N, 1), f32),
            jax.ShapeDtypeStruct((N, 64), f32),
        ],
    )(h1, degp)


def _tc_mid(p, dinv, b, Wn, relu, leaf_cols, leaf_from_act=False):
    fo = p.shape[2]
    fn = Wn.shape[1]

    def body(p_ref, dinv_ref, b_ref, w_ref, *out_refs):
        dinv = dinv_ref[...]
        out = dinv * (p_ref[0] + p_ref[1]) + b_ref[...]
        act = jnp.maximum(out, 0.0) if relu else out
        if leaf_cols:
            leaf = act if leaf_from_act else out
            out_refs[0][...] = leaf[:, :leaf_cols]
        h = jnp.dot(act, w_ref[...], preferred_element_type=f32)
        out_refs[-1][...] = h * dinv

    out_specs = [pl.BlockSpec((BN, fn), lambda i: (i, 0))]
    out_shape = [jax.ShapeDtypeStruct((N, fn), f32)]
    if leaf_cols:
        out_specs.insert(0, pl.BlockSpec((BN, leaf_cols), lambda i: (i, 0)))
        out_shape.insert(0, jax.ShapeDtypeStruct((N, leaf_cols), f32))

    res = pl.pallas_call(
        body,
        grid=(N // BN,),
        in_specs=[
            pl.BlockSpec((NC, BN, fo), lambda i: (0, i, 0)),
            pl.BlockSpec((BN, 1), lambda i: (i, 0)),
            pl.BlockSpec((1, fo), lambda i: (0, 0)),
            pl.BlockSpec(Wn.shape, lambda i: (0, 0)),
        ],
        out_specs=out_specs,
        out_shape=out_shape,
    )(p, dinv, b, Wn)
    return res if leaf_cols else (None, res[0])


def _tc_last(p, dinv, b):
    fo = p.shape[2]

    def body(p_ref, dinv_ref, b_ref, lp_ref):
        out = dinv_ref[...] * (p_ref[0] + p_ref[1]) + b_ref[...]
        x9 = out[:, :3]
        m = jnp.max(x9, axis=1, keepdims=True)
        e = jnp.exp(x9 - m)
        lp_ref[...] = x9 - m - jnp.log(jnp.sum(e, axis=1, keepdims=True))

    return pl.pallas_call(
        body,
        grid=(N // BN,),
        in_specs=[
            pl.BlockSpec((NC, BN, fo), lambda i: (0, i, 0)),
            pl.BlockSpec((BN, 1), lambda i: (i, 0)),
            pl.BlockSpec((1, fo), lambda i: (0, 0)),
        ],
        out_specs=pl.BlockSpec((BN, 3), lambda i: (i, 0)),
        out_shape=jax.ShapeDtypeStruct((N, 3), f32),
    )(p, dinv, b)


def kernel(x, edge_index, W1, b1, W2, b2, W3, b3, W4, b4, W5, b5):
    src3 = edge_index[0].reshape(NW, CH, CB)
    dst3 = edge_index[1].reshape(NW, CH, CB)

    W3p = jnp.pad(W3, ((0, 0), (0, 8)))
    b3p = jnp.pad(b3, (0, 8))
    W4p = jnp.pad(W4, ((0, 8), (0, 0)))
    W5p = jnp.pad(W5, ((0, 0), (0, 13)))
    b5p = jnp.pad(b5, (0, 13))
    ones16 = jnp.ones((N, 16), f32)
    z16 = jnp.zeros((N, 16), f32)
    z32 = jnp.zeros((N, 32), f32)
    z64 = jnp.zeros((N, 64), f32)

    h1 = _tc_matmul1(x, W1)
    degp = _make_agg(16, gather=False)(ones16, z16, src3, dst3)
    dinv, hs1 = _tc_first(h1, degp)

    p1 = _make_agg(64, gather=True)(hs1, z64, src3, dst3)
    x2, hs2 = _tc_mid(p1, dinv, b1.reshape(1, -1), W2,
                      relu=True, leaf_cols=64, leaf_from_act=True)

    p2 = _make_agg(32, gather=True)(hs2, z32, src3, dst3)
    _, hs3 = _tc_mid(p2, dinv, b2.reshape(1, -1), W3p,
                     relu=True, leaf_cols=0)

    p3 = _make_agg(32, gather=True)(hs3, z32, src3, dst3)
    x5, hs4 = _tc_mid(p3, dinv, b3p.reshape(1, -1), W4p,
                      relu=True, leaf_cols=24)

    p4 = _make_agg(16, gather=True)(hs4, z16, src3, dst3)
    x8, hs5 = _tc_mid(p4, dinv, b4.reshape(1, -1), W5p,
                      relu=False, leaf_cols=16)

    p5 = _make_agg(16, gather=True)(hs5, z16, src3, dst3)
    logp = _tc_last(p5, dinv, b5p.reshape(1, -1))

    return (logp, x2, x5, x8)
